# baseline (speedup 1.0000x reference)
import functools

import jax
import jax.numpy as jnp
from jax import lax
from jax.experimental import pallas as pl
from jax.experimental.pallas import tpu as pltpu
from jax.experimental.pallas import tpu_sc as plsc

N = 10000
E = 320000
NC = 2
NS = 16
EB = 128
BPW = 80
HB = 40
EPAD = NC * NS * BPW * EB
NPAD = 10240
RPT = NPAD // NS


_GDN = lax.GatherDimensionNumbers(
    offset_dims=(), collapsed_slice_dims=(0,), start_index_map=(0,))


def _lane_broadcast(vec, lane):
    idx = jnp.full((16, 1), lane, jnp.int32)
    return lax.gather(vec, idx, _GDN, slice_sizes=(1,),
                      mode=lax.GatherScatterMode.PROMISE_IN_BOUNDS)


def _make_spmm(W):
    FC = W // 16

    mesh = plsc.VectorSubcoreMesh(core_axis_name="c", subcore_axis_name="s")

    scratch = [
        pltpu.VMEM((HB, EB), jnp.int32),
        pltpu.VMEM((HB, EB), jnp.int32),
        pltpu.VMEM((HB, EB), jnp.float32),
        pltpu.VMEM_SHARED((NPAD, W), jnp.float32),
        pltpu.VMEM((EB, W), jnp.float32),
        pltpu.VMEM((EB, W), jnp.float32),
        pltpu.SemaphoreType.DMA,
        pltpu.SemaphoreType.DMA,
        pltpu.SemaphoreType.DMA,
        pltpu.SemaphoreType.DMA,
    ]

    @functools.partial(
        pl.kernel,
        out_type=jax.ShapeDtypeStruct((NC, NPAD, W), jnp.float32),
        mesh=mesh,
        scratch_types=scratch,
    )
    def spmm(h_hbm, src_hbm, dst_hbm, ew_hbm, zero_hbm, out_hbm,
             srcv, dstv, wv, acc, r0, r1, g0, g1, s0, s1):
        rows = [r0, r1]
        gsem = [g0, g1]
        ssem = [s0, s1]
        cid = lax.axis_index("c")
        sid = lax.axis_index("s")
        wid = cid * NS + sid

        pltpu.sync_copy(zero_hbm, acc.at[pl.ds(sid * RPT, RPT)])
        plsc.subcore_barrier()

        def start_gather(j, b):
            pltpu.async_copy(h_hbm.at[srcv.at[j]], rows[b], gsem[b])

        def wait_gather(j, b):
            pltpu.make_async_copy(h_hbm.at[srcv.at[j]], rows[b], gsem[b]).wait()

        def start_scatter(j, b):
            pltpu.async_copy(rows[b], acc.at[dstv.at[j]], ssem[b], add=True)

        def wait_scatter(j, b):
            pltpu.make_async_copy(rows[b], acc.at[dstv.at[j]], ssem[b]).wait()

        def scale(j, b):
            def grp(g, c2):
                wg = wv[j, pl.ds(g * 16, 16)]
                for i in range(16):
                    ws = _lane_broadcast(wg, i)
                    e = g * 16 + i
                    for f in range(FC):
                        sl = pl.ds(f * 16, 16)
                        rows[b][e, sl] = rows[b][e, sl] * ws
                return c2

            lax.fori_loop(0, EB // 16, grp, 0)

        for h in range(BPW // HB):
            cbase = wid * BPW + h * HB
            pltpu.sync_copy(src_hbm.at[pl.ds(cbase, HB)], srcv)
            pltpu.sync_copy(dst_hbm.at[pl.ds(cbase, HB)], dstv)
            pltpu.sync_copy(ew_hbm.at[pl.ds(cbase, HB)], wv)

            start_gather(0, 0)

            def outer(jj, carry):
                for b in range(2):
                    j = jj * 2 + b
                    wait_gather(j, b)
                    scale(j, b)
                    if b == 0:
                        @pl.when(jj >= 1)
                        def _():
                            wait_scatter(j - 1, 1)
                        start_scatter(j, b)
                        start_gather(j + 1, 1)
                    else:
                        wait_scatter(j - 1, 0)
                        start_scatter(j, b)

                        @pl.when(jj < HB // 2 - 1)
                        def _():
                            start_gather(j + 1, 0)
                return carry

            lax.fori_loop(0, HB // 2, outer, 0)
            wait_scatter(HB - 1, 1)

        plsc.subcore_barrier()
        rr = sid * RPT
        pltpu.sync_copy(acc.at[pl.ds(rr, RPT)], out_hbm.at[cid, pl.ds(rr, RPT)])

    return spmm


_spmm128 = _make_spmm(128)

_BM = 1000


def _mm_body(x_ref, w_ref, o_ref):
    o_ref[...] = jnp.dot(x_ref[...], w_ref[...],
                         preferred_element_type=jnp.float32)


def _matmul_tc(x, w):
    m, k = x.shape
    n = w.shape[1]
    return pl.pallas_call(
        _mm_body,
        grid=(m // _BM,),
        in_specs=[pl.BlockSpec((_BM, k), lambda i: (i, 0)),
                  pl.BlockSpec((k, n), lambda i: (0, 0))],
        out_specs=pl.BlockSpec((_BM, n), lambda i: (i, 0)),
        out_shape=jax.ShapeDtypeStruct((m, n), jnp.float32),
    )(x, w)


def _mid_body(p0_ref, p1_ref, b_ref, w_ref, o_ref):
    h = jnp.maximum(p0_ref[...] + p1_ref[...] + b_ref[...], 0.0)
    o_ref[...] = jnp.dot(h, w_ref[...], preferred_element_type=jnp.float32)


def _mid_tc(p0, p1, b1, w2):
    m, k = p0.shape
    n = w2.shape[1]
    return pl.pallas_call(
        _mid_body,
        grid=(m // _BM,),
        in_specs=[pl.BlockSpec((_BM, k), lambda i: (i, 0)),
                  pl.BlockSpec((_BM, k), lambda i: (i, 0)),
                  pl.BlockSpec((1, k), lambda i: (0, 0)),
                  pl.BlockSpec((k, n), lambda i: (0, 0))],
        out_specs=pl.BlockSpec((_BM, n), lambda i: (i, 0)),
        out_shape=jax.ShapeDtypeStruct((m, n), jnp.float32),
    )(p0, p1, b1, w2)


def _final_body(q0_ref, q1_ref, b_ref, o_ref):
    z = q0_ref[...] + q1_ref[...] + b_ref[...]
    z = z - jnp.max(z, axis=1, keepdims=True)
    o_ref[...] = z - jnp.log(jnp.sum(jnp.exp(z), axis=1, keepdims=True))


def _final_tc(q0, q1, b2):
    m, n = q0.shape
    return pl.pallas_call(
        _final_body,
        grid=(m // _BM,),
        in_specs=[pl.BlockSpec((_BM, n), lambda i: (i, 0)),
                  pl.BlockSpec((_BM, n), lambda i: (i, 0)),
                  pl.BlockSpec((1, n), lambda i: (0, 0))],
        out_specs=pl.BlockSpec((_BM, n), lambda i: (i, 0)),
        out_shape=jax.ShapeDtypeStruct((m, n), jnp.float32),
    )(q0, q1, b2)


def kernel(x, edge_index, edge_weight, W1, b1, W2, b2):
    pad = EPAD - edge_index.shape[1]
    spread = jnp.arange(pad, dtype=jnp.int32) % N
    src = jnp.concatenate([edge_index[0], spread]).reshape(-1, EB)
    dst = jnp.concatenate([edge_index[1], spread]).reshape(-1, EB)
    ew = jnp.concatenate([edge_weight,
                          jnp.zeros((pad,), jnp.float32)]).reshape(-1, EB)
    zeros = jnp.zeros((RPT, 128), jnp.float32)
    h1 = _matmul_tc(x, W1)
    p = _spmm128(h1, src, dst, ew, zeros)
    w2p = jnp.concatenate([W2, jnp.zeros((W2.shape[0], 128 - W2.shape[1]),
                                         jnp.float32)], axis=1)
    h2 = _mid_tc(p[0, :N], p[1, :N], b1.reshape(1, -1), w2p)
    q = _spmm128(h2, src, dst, ew, zeros)
    ncls = W2.shape[1]
    return _final_tc(q[0, :N, :ncls], q[1, :N, :ncls], b2.reshape(1, -1))


# speedup vs baseline: 1.2986x; 1.2986x over previous
import functools

import jax
import jax.numpy as jnp
from jax import lax
from jax.experimental import pallas as pl
from jax.experimental.pallas import tpu as pltpu
from jax.experimental.pallas import tpu_sc as plsc

N = 10000
E = 320000
NC = 2
NS = 16
EB = 128
BPW = 84
EPAD = NC * NS * BPW * EB
CHK = 2
K12 = 12
NSB = BPW // K12
SR = 624


_GDN = lax.GatherDimensionNumbers(
    offset_dims=(), collapsed_slice_dims=(0,), start_index_map=(0,))


def _lane_broadcast(vec, lane):
    idx = jnp.full((16, 1), lane, jnp.int32)
    return lax.gather(vec, idx, _GDN, slice_sizes=(1,),
                      mode=lax.GatherScatterMode.PROMISE_IN_BOUNDS)


def _make_spmm(W):
    FC = W // 16

    mesh = plsc.VectorSubcoreMesh(core_axis_name="c", subcore_axis_name="s")

    scratch = [
        pltpu.VMEM((2 * CHK, EB), jnp.int32),
        pltpu.VMEM((2 * CHK, EB), jnp.int32),
        pltpu.VMEM((2 * CHK, EB), jnp.float32),
        pltpu.VMEM_SHARED((N, W), jnp.float32),
        pltpu.VMEM((EB, W), jnp.float32),
        pltpu.VMEM((EB, W), jnp.float32),
        pltpu.VMEM((EB, W), jnp.float32),
    ]
    scratch += [pltpu.SemaphoreType.DMA] * 12

    @functools.partial(
        pl.kernel,
        out_type=jax.ShapeDtypeStruct((NC, N, W), jnp.float32),
        mesh=mesh,
        scratch_types=scratch,
    )
    def spmm(h_hbm, src_hbm, dst_hbm, ew_hbm, zero_hbm, out_hbm,
             srcv, dstv, wv, acc, r0, r1, r2,
             g0, g1, g2, s0, s1, s2, si0, si1, wi0, wi1, di0, di1):
        rows = [r0, r1, r2]
        gsem = [g0, g1, g2]
        ssem = [s0, s1, s2]
        sisem = [si0, si1]
        wisem = [wi0, wi1]
        disem = [di0, di1]
        cid = lax.axis_index("c")
        sid = lax.axis_index("s")
        wid = cid * NS + sid
        wbase = wid * BPW

        pltpu.sync_copy(zero_hbm, acc.at[pl.ds(sid * SR, SR)])

        @pl.when(sid == NS - 1)
        def _():
            pltpu.sync_copy(zero_hbm.at[pl.ds(0, 16)],
                            acc.at[pl.ds(NS * SR, 16)])

        plsc.subcore_barrier()

        def start_gather(j, k):
            kc = k // 2
            pltpu.async_copy(h_hbm.at[srcv.at[2 * (kc % 2) + k % 2]],
                             rows[k % 3], gsem[k % 3])

        def wait_gather(j, k):
            kc = k // 2
            pltpu.make_async_copy(h_hbm.at[srcv.at[2 * (kc % 2) + k % 2]],
                                  rows[k % 3], gsem[k % 3]).wait()

        def start_scatter(j, k):
            kc = k // 2
            pltpu.async_copy(rows[k % 3], acc.at[dstv.at[2 * (kc % 2) + k % 2]],
                             ssem[k % 3], add=True)

        def wait_scatter(j, k):
            kc = k // 2
            pltpu.make_async_copy(rows[k % 3],
                                  acc.at[dstv.at[2 * (kc % 2) + k % 2]],
                                  ssem[k % 3]).wait()

        def prefetch_sw(c, kc):
            s2_ = 2 * (kc % 2)
            pltpu.async_copy(src_hbm.at[pl.ds(wbase + CHK * c, CHK)],
                             srcv.at[pl.ds(s2_, CHK)], sisem[kc % 2])
            pltpu.async_copy(ew_hbm.at[pl.ds(wbase + CHK * c, CHK)],
                             wv.at[pl.ds(s2_, CHK)], wisem[kc % 2])

        def prefetch_d(c, kc):
            d2_ = 2 * (kc % 2)
            pltpu.async_copy(dst_hbm.at[pl.ds(wbase + CHK * c, CHK)],
                             dstv.at[pl.ds(d2_, CHK)], disem[kc % 2])

        def wait_src(c, kc):
            s2_ = 2 * (kc % 2)
            pltpu.make_async_copy(src_hbm.at[pl.ds(wbase + CHK * c, CHK)],
                                  srcv.at[pl.ds(s2_, CHK)],
                                  sisem[kc % 2]).wait()

        def wait_w(c, kc):
            s2_ = 2 * (kc % 2)
            pltpu.make_async_copy(ew_hbm.at[pl.ds(wbase + CHK * c, CHK)],
                                  wv.at[pl.ds(s2_, CHK)],
                                  wisem[kc % 2]).wait()

        def wait_dst(c, kc):
            d2_ = 2 * (kc % 2)
            pltpu.make_async_copy(dst_hbm.at[pl.ds(wbase + CHK * c, CHK)],
                                  dstv.at[pl.ds(d2_, CHK)],
                                  disem[kc % 2]).wait()

        def scale(j, k):
            kc = k // 2
            wrow = 2 * (kc % 2) + k % 2
            buf = rows[k % 3]

            def grp(g, c2):
                wg = wv[wrow, pl.ds(g * 16, 16)]
                for i in range(16):
                    ws = _lane_broadcast(wg, i)
                    e = g * 16 + i
                    for f in range(FC):
                        sl = pl.ds(f * 16, 16)
                        buf[e, sl] = buf[e, sl] * ws
                return c2

            lax.fori_loop(0, EB // 16, grp, 0)

        for c0 in range(2):
            pltpu.sync_copy(src_hbm.at[pl.ds(wbase + CHK * c0, CHK)],
                            srcv.at[pl.ds(2 * c0, CHK)])
            pltpu.sync_copy(dst_hbm.at[pl.ds(wbase + CHK * c0, CHK)],
                            dstv.at[pl.ds(2 * c0, CHK)])
            pltpu.sync_copy(ew_hbm.at[pl.ds(wbase + CHK * c0, CHK)],
                            wv.at[pl.ds(2 * c0, CHK)])
        start_gather(0, 0)

        def outer(jj, carry):
            for k in range(K12):
                j = jj * K12 + k
                kc = k // 2
                c = j // 2
                if k >= 2:
                    wait_scatter(j - 2, k - 2)
                else:
                    @pl.when(jj >= 1)
                    def _():
                        wait_scatter(j - 2, k - 2 + K12)
                if k % 2 == 0:
                    if k >= 4:
                        wait_w(c, kc)
                    else:
                        @pl.when(jj >= 1)
                        def _():
                            wait_w(c, kc)
                    start_gather(j + 1, k + 1)
                    wait_gather(j, k)
                    scale(j, k)
                    if k >= 4:
                        wait_dst(c, kc)
                    else:
                        @pl.when(jj >= 1)
                        def _():
                            wait_dst(c, kc)
                    start_scatter(j, k)
                else:
                    if k == 1:
                        @pl.when(jj >= 1)
                        def _():
                            prefetch_d(c + 1, kc + 1)
                            wait_src(c + 1, kc + 1)
                        start_gather(j + 1, k + 1)
                    elif k < K12 - 1:
                        prefetch_d(c + 1, kc + 1)
                        wait_src(c + 1, kc + 1)
                        start_gather(j + 1, k + 1)
                    else:
                        @pl.when(jj < NSB - 1)
                        def _():
                            prefetch_d(c + 1, kc + 1)
                            wait_src(c + 1, kc + 1)
                            start_gather(j + 1, k + 1)
                    wait_gather(j, k)
                    scale(j, k)
                    start_scatter(j, k)
                    @pl.when(j < BPW - 4)
                    def _():
                        prefetch_sw(c + 2, kc + 2)
            return carry

        lax.fori_loop(0, NSB, outer, 0)
        wait_scatter(BPW - 2, K12 - 2)
        wait_scatter(BPW - 1, K12 - 1)

        plsc.subcore_barrier()
        rr = sid * SR
        pltpu.sync_copy(acc.at[pl.ds(rr, SR)], out_hbm.at[cid, pl.ds(rr, SR)])

        @pl.when(sid == NS - 1)
        def _():
            pltpu.sync_copy(acc.at[pl.ds(NS * SR, 16)],
                            out_hbm.at[cid, pl.ds(NS * SR, 16)])

    return spmm


_spmm128 = _make_spmm(128)

_BM = 1000


def _mm_body(x_ref, w_ref, o_ref):
    o_ref[...] = jnp.dot(x_ref[...], w_ref[...],
                         preferred_element_type=jnp.float32)


def _matmul_tc(x, w):
    m, k = x.shape
    n = w.shape[1]
    return pl.pallas_call(
        _mm_body,
        grid=(m // _BM,),
        in_specs=[pl.BlockSpec((_BM, k), lambda i: (i, 0)),
                  pl.BlockSpec((k, n), lambda i: (0, 0))],
        out_specs=pl.BlockSpec((_BM, n), lambda i: (i, 0)),
        out_shape=jax.ShapeDtypeStruct((m, n), jnp.float32),
    )(x, w)


def _mid_body(p0_ref, p1_ref, b_ref, w_ref, o_ref):
    h = jnp.maximum(p0_ref[...] + p1_ref[...] + b_ref[...], 0.0)
    o_ref[...] = jnp.dot(h, w_ref[...], preferred_element_type=jnp.float32)


def _mid_tc(p0, p1, b1, w2):
    m, k = p0.shape
    n = w2.shape[1]
    return pl.pallas_call(
        _mid_body,
        grid=(m // _BM,),
        in_specs=[pl.BlockSpec((_BM, k), lambda i: (i, 0)),
                  pl.BlockSpec((_BM, k), lambda i: (i, 0)),
                  pl.BlockSpec((1, k), lambda i: (0, 0)),
                  pl.BlockSpec((k, n), lambda i: (0, 0))],
        out_specs=pl.BlockSpec((_BM, n), lambda i: (i, 0)),
        out_shape=jax.ShapeDtypeStruct((m, n), jnp.float32),
    )(p0, p1, b1, w2)


def _final_body(q0_ref, q1_ref, b_ref, o_ref):
    z = q0_ref[...] + q1_ref[...] + b_ref[...]
    z = z - jnp.max(z, axis=1, keepdims=True)
    o_ref[...] = z - jnp.log(jnp.sum(jnp.exp(z), axis=1, keepdims=True))


def _final_tc(q0, q1, b2):
    m, n = q0.shape
    return pl.pallas_call(
        _final_body,
        grid=(m // _BM,),
        in_specs=[pl.BlockSpec((_BM, n), lambda i: (i, 0)),
                  pl.BlockSpec((_BM, n), lambda i: (i, 0)),
                  pl.BlockSpec((1, n), lambda i: (0, 0))],
        out_specs=pl.BlockSpec((_BM, n), lambda i: (i, 0)),
        out_shape=jax.ShapeDtypeStruct((m, n), jnp.float32),
    )(q0, q1, b2)


def kernel(x, edge_index, edge_weight, W1, b1, W2, b2):
    pad = EPAD - edge_index.shape[1]
    spread = jnp.arange(pad, dtype=jnp.int32) % N
    src = jnp.concatenate([edge_index[0], spread]).reshape(-1, EB)
    dst = jnp.concatenate([edge_index[1], spread]).reshape(-1, EB)
    ew = jnp.concatenate([edge_weight,
                          jnp.zeros((pad,), jnp.float32)]).reshape(-1, EB)
    zeros = jnp.zeros((SR, 128), jnp.float32)
    h1 = _matmul_tc(x, W1)
    p = _spmm128(h1, src, dst, ew, zeros)
    w2p = jnp.concatenate([W2, jnp.zeros((W2.shape[0], 128 - W2.shape[1]),
                                         jnp.float32)], axis=1)
    h2 = _mid_tc(p[0], p[1], b1.reshape(1, -1), w2p)
    q = _spmm128(h2, src, dst, ew, zeros)
    ncls = W2.shape[1]
    return _final_tc(q[0, :, :ncls], q[1, :, :ncls], b2.reshape(1, -1))


# speedup vs baseline: 1.3331x; 1.0266x over previous
import functools

import jax
import jax.numpy as jnp
from jax import lax
from jax.experimental import pallas as pl
from jax.experimental.pallas import tpu as pltpu
from jax.experimental.pallas import tpu_sc as plsc

N = 10000
E = 320000
NC = 2
NS = 16
EB = 128
BPW = 84
EPAD = NC * NS * BPW * EB
CHK = 2
K12 = 12
NSB = BPW // K12
SR = 624


_GDN = lax.GatherDimensionNumbers(
    offset_dims=(), collapsed_slice_dims=(0,), start_index_map=(0,))


def _lane_broadcast(vec, lane):
    idx = jnp.full((16, 1), lane, jnp.int32)
    return lax.gather(vec, idx, _GDN, slice_sizes=(1,),
                      mode=lax.GatherScatterMode.PROMISE_IN_BOUNDS)


def _make_spmm(W):
    FC = W // 16

    mesh = plsc.VectorSubcoreMesh(core_axis_name="c", subcore_axis_name="s")

    scratch = [
        pltpu.VMEM((2 * CHK, EB), jnp.int32),
        pltpu.VMEM((2 * CHK, EB), jnp.int32),
        pltpu.VMEM((2 * CHK, EB), jnp.float32),
        pltpu.VMEM_SHARED((N, W), jnp.float32),
        pltpu.VMEM((EB, W), jnp.float32),
        pltpu.VMEM((EB, W), jnp.float32),
        pltpu.VMEM((EB, W), jnp.float32),
    ]
    scratch += [pltpu.SemaphoreType.DMA] * 12

    @functools.partial(
        pl.kernel,
        out_type=jax.ShapeDtypeStruct((NC, N, W), jnp.float32),
        mesh=mesh,
        scratch_types=scratch,
    )
    def spmm(h_hbm, src_hbm, dst_hbm, ew_hbm, zero_hbm, out_hbm,
             srcv, dstv, wv, acc, r0, r1, r2,
             g0, g1, g2, s0, s1, s2, si0, si1, wi0, wi1, di0, di1):
        rows = [r0, r1, r2]
        gsem = [g0, g1, g2]
        ssem = [s0, s1, s2]
        sisem = [si0, si1]
        wisem = [wi0, wi1]
        disem = [di0, di1]
        cid = lax.axis_index("c")
        sid = lax.axis_index("s")
        wid = cid * NS + sid
        wbase = wid * BPW

        pltpu.sync_copy(zero_hbm, acc.at[pl.ds(sid * SR, SR)])

        @pl.when(sid == NS - 1)
        def _():
            pltpu.sync_copy(zero_hbm.at[pl.ds(0, 16)],
                            acc.at[pl.ds(NS * SR, 16)])

        plsc.subcore_barrier()

        def start_gather(j, k):
            kc = k // 2
            pltpu.async_copy(h_hbm.at[srcv.at[2 * (kc % 2) + k % 2]],
                             rows[k % 3], gsem[k % 3])

        def wait_gather(j, k):
            kc = k // 2
            pltpu.make_async_copy(h_hbm.at[srcv.at[2 * (kc % 2) + k % 2]],
                                  rows[k % 3], gsem[k % 3]).wait()

        def start_scatter(j, k):
            kc = k // 2
            pltpu.async_copy(rows[k % 3], acc.at[dstv.at[2 * (kc % 2) + k % 2]],
                             ssem[k % 3], add=True)

        def wait_scatter(j, k):
            kc = k // 2
            pltpu.make_async_copy(rows[k % 3],
                                  acc.at[dstv.at[2 * (kc % 2) + k % 2]],
                                  ssem[k % 3]).wait()

        def prefetch_sw(c, kc):
            s2_ = 2 * (kc % 2)
            pltpu.async_copy(src_hbm.at[pl.ds(wbase + CHK * c, CHK)],
                             srcv.at[pl.ds(s2_, CHK)], sisem[kc % 2])
            pltpu.async_copy(ew_hbm.at[pl.ds(wbase + CHK * c, CHK)],
                             wv.at[pl.ds(s2_, CHK)], wisem[kc % 2])

        def prefetch_d(c, kc):
            d2_ = 2 * (kc % 2)
            pltpu.async_copy(dst_hbm.at[pl.ds(wbase + CHK * c, CHK)],
                             dstv.at[pl.ds(d2_, CHK)], disem[kc % 2])

        def wait_src(c, kc):
            s2_ = 2 * (kc % 2)
            pltpu.make_async_copy(src_hbm.at[pl.ds(wbase + CHK * c, CHK)],
                                  srcv.at[pl.ds(s2_, CHK)],
                                  sisem[kc % 2]).wait()

        def wait_w(c, kc):
            s2_ = 2 * (kc % 2)
            pltpu.make_async_copy(ew_hbm.at[pl.ds(wbase + CHK * c, CHK)],
                                  wv.at[pl.ds(s2_, CHK)],
                                  wisem[kc % 2]).wait()

        def wait_dst(c, kc):
            d2_ = 2 * (kc % 2)
            pltpu.make_async_copy(dst_hbm.at[pl.ds(wbase + CHK * c, CHK)],
                                  dstv.at[pl.ds(d2_, CHK)],
                                  disem[kc % 2]).wait()

        def scale(j, k):
            kc = k // 2
            wrow = 2 * (kc % 2) + k % 2
            buf = rows[k % 3]

            def grp(g, c2):
                wg = wv[wrow, pl.ds(g * 16, 16)]
                for i in range(16):
                    ws = _lane_broadcast(wg, i)
                    e = g * 16 + i
                    for f in range(FC):
                        sl = pl.ds(f * 16, 16)
                        buf[e, sl] = buf[e, sl] * ws
                return c2

            lax.fori_loop(0, EB // 16, grp, 0)

        for c0 in range(2):
            pltpu.sync_copy(src_hbm.at[pl.ds(wbase + CHK * c0, CHK)],
                            srcv.at[pl.ds(2 * c0, CHK)])
            pltpu.sync_copy(dst_hbm.at[pl.ds(wbase + CHK * c0, CHK)],
                            dstv.at[pl.ds(2 * c0, CHK)])
            pltpu.sync_copy(ew_hbm.at[pl.ds(wbase + CHK * c0, CHK)],
                            wv.at[pl.ds(2 * c0, CHK)])
        start_gather(0, 0)

        def outer(jj, carry):
            for k in range(K12):
                j = jj * K12 + k
                kc = k // 2
                c = j // 2
                if k >= 2:
                    wait_scatter(j - 2, k - 2)
                else:
                    @pl.when(jj >= 1)
                    def _():
                        wait_scatter(j - 2, k - 2 + K12)
                if k % 2 == 0:
                    if k >= 4:
                        wait_w(c, kc)
                    else:
                        @pl.when(jj >= 1)
                        def _():
                            wait_w(c, kc)
                    start_gather(j + 1, k + 1)
                    wait_gather(j, k)
                    scale(j, k)
                    if k >= 4:
                        wait_dst(c, kc)
                    else:
                        @pl.when(jj >= 1)
                        def _():
                            wait_dst(c, kc)
                    start_scatter(j, k)
                else:
                    if k == 1:
                        @pl.when(jj >= 1)
                        def _():
                            prefetch_d(c + 1, kc + 1)
                            wait_src(c + 1, kc + 1)
                        start_gather(j + 1, k + 1)
                    elif k < K12 - 1:
                        prefetch_d(c + 1, kc + 1)
                        wait_src(c + 1, kc + 1)
                        start_gather(j + 1, k + 1)
                    else:
                        @pl.when(jj < NSB - 1)
                        def _():
                            prefetch_d(c + 1, kc + 1)
                            wait_src(c + 1, kc + 1)
                            start_gather(j + 1, k + 1)
                    wait_gather(j, k)
                    scale(j, k)
                    start_scatter(j, k)
                    @pl.when(j < BPW - 4)
                    def _():
                        prefetch_sw(c + 2, kc + 2)
            return carry

        lax.fori_loop(0, NSB, outer, 0)
        wait_scatter(BPW - 2, K12 - 2)
        wait_scatter(BPW - 1, K12 - 1)

        plsc.subcore_barrier()
        rr = sid * SR
        pltpu.sync_copy(acc.at[pl.ds(rr, SR)], out_hbm.at[cid, pl.ds(rr, SR)])

        @pl.when(sid == NS - 1)
        def _():
            pltpu.sync_copy(acc.at[pl.ds(NS * SR, 16)],
                            out_hbm.at[cid, pl.ds(NS * SR, 16)])

    return spmm


_spmm128 = _make_spmm(128)

_BM = 1000


def _mid_body(p0_ref, p1_ref, b_ref, w_ref, o_ref):
    s = p0_ref[...] + p1_ref[...]
    h = jnp.dot(s, w_ref[...], preferred_element_type=jnp.float32)
    o_ref[...] = jnp.maximum(h + b_ref[...], 0.0)


def _mid_tc(p0, p1, b1, w1):
    m, k = p0.shape
    n = w1.shape[1]
    return pl.pallas_call(
        _mid_body,
        grid=(m // _BM,),
        in_specs=[pl.BlockSpec((_BM, k), lambda i: (i, 0)),
                  pl.BlockSpec((_BM, k), lambda i: (i, 0)),
                  pl.BlockSpec((1, n), lambda i: (0, 0)),
                  pl.BlockSpec((k, n), lambda i: (0, 0))],
        out_specs=pl.BlockSpec((_BM, n), lambda i: (i, 0)),
        out_shape=jax.ShapeDtypeStruct((m, n), jnp.float32),
    )(p0, p1, b1, w1)


def _final_body(q0_ref, q1_ref, b_ref, w_ref, o_ref):
    s = q0_ref[...] + q1_ref[...]
    z = jnp.dot(s, w_ref[...], preferred_element_type=jnp.float32)
    z = z + b_ref[...]
    z = z - jnp.max(z, axis=1, keepdims=True)
    o_ref[...] = z - jnp.log(jnp.sum(jnp.exp(z), axis=1, keepdims=True))


def _final_tc(q0, q1, b2, w2):
    m, k = q0.shape
    n = w2.shape[1]
    return pl.pallas_call(
        _final_body,
        grid=(m // _BM,),
        in_specs=[pl.BlockSpec((_BM, k), lambda i: (i, 0)),
                  pl.BlockSpec((_BM, k), lambda i: (i, 0)),
                  pl.BlockSpec((1, n), lambda i: (0, 0)),
                  pl.BlockSpec((k, n), lambda i: (0, 0))],
        out_specs=pl.BlockSpec((_BM, n), lambda i: (i, 0)),
        out_shape=jax.ShapeDtypeStruct((m, n), jnp.float32),
    )(q0, q1, b2, w2)


def kernel(x, edge_index, edge_weight, W1, b1, W2, b2):
    pad = EPAD - edge_index.shape[1]
    spread = jnp.arange(pad, dtype=jnp.int32) % N
    src = jnp.concatenate([edge_index[0], spread]).reshape(-1, EB)
    dst = jnp.concatenate([edge_index[1], spread]).reshape(-1, EB)
    ew = jnp.concatenate([edge_weight,
                          jnp.zeros((pad,), jnp.float32)]).reshape(-1, EB)
    zeros = jnp.zeros((SR, 128), jnp.float32)
    p = _spmm128(x, src, dst, ew, zeros)
    h = _mid_tc(p[0], p[1], b1.reshape(1, -1), W1)
    q = _spmm128(h, src, dst, ew, zeros)
    return _final_tc(q[0], q[1], b2.reshape(1, -1), W2)
